# baseline (speedup 1.0000x reference)
import functools

import jax
import jax.numpy as jnp
from jax import lax
from jax.experimental import pallas as pl
from jax.experimental.pallas import tpu as pltpu
from jax.experimental.pallas import tpu_sc as plsc

B_, S_, D_, R_ = 1, 2048, 768, 256
E_, K_, H_ = 64, 2, 32
DH_ = D_ // H_
FFN_ = 3072
T_ = B_ * S_
C_ = 80
CP_ = 88
TB_ = 128
NB_ = T_ // TB_


def _router_body(x_ref, win_ref, bin_ref, bmat_ref, wg_ref, bg_ref,
                 g0_ref, g1_ref, srctok_ref, wslot_ref,
                 carry_ref, accs_ref, accw_ref):
    pid = pl.program_id(0)

    @pl.when(pid == 0)
    def _init():
        carry_ref[...] = jnp.zeros_like(carry_ref)
        accs_ref[...] = jnp.zeros_like(accs_ref)
        accw_ref[...] = jnp.zeros_like(accw_ref)

    xb = x_ref[...]
    xp = jnp.dot(xb, win_ref[...], preferred_element_type=jnp.float32) + bin_ref[...]
    h = 0.1 * jnp.tanh(jnp.dot(xp, bmat_ref[...], preferred_element_type=jnp.float32))
    logits = jnp.dot(h, wg_ref[...], preferred_element_type=jnp.float32) + bg_ref[...]

    iota_e = lax.broadcasted_iota(jnp.int32, (TB_, E_), 1)
    m1 = jnp.max(logits, axis=1, keepdims=True)
    i1 = jnp.min(jnp.where(logits == m1, iota_e, E_), axis=1)
    masked = jnp.where(iota_e == i1[:, None], jnp.float32(-jnp.inf), logits)
    m2 = jnp.max(masked, axis=1, keepdims=True)
    i2 = jnp.min(jnp.where(masked == m2, iota_e, E_), axis=1)

    t = jnp.exp(m2 - m1)
    wa = 1.0 / (1.0 + t)
    wb = t / (1.0 + t)

    oh0 = (iota_e == i1[:, None]).astype(jnp.float32)
    oh1 = (iota_e == i2[:, None]).astype(jnp.float32)
    both = oh0 + oh1
    r_i = lax.broadcasted_iota(jnp.int32, (TB_, TB_), 0)
    c_i = lax.broadcasted_iota(jnp.int32, (TB_, TB_), 1)
    ltri = (c_i < r_i).astype(jnp.float32)
    sexc = jnp.dot(ltri, both, preferred_element_type=jnp.float32)
    base = carry_ref[...] + sexc
    pos0 = jnp.sum(oh0 * base, axis=1)
    pos1 = jnp.sum(oh1 * base, axis=1)
    carry_ref[...] = carry_ref[...] + jnp.sum(both, axis=0, keepdims=True)

    keep0 = (pos0 < C_).astype(jnp.float32)
    keep1 = (pos1 < C_).astype(jnp.float32)
    p0c = jnp.minimum(pos0, C_ - 1).astype(jnp.int32)
    p1c = jnp.minimum(pos1, C_ - 1).astype(jnp.int32)
    weff0 = wa[:, 0] * keep0
    weff1 = wb[:, 0] * keep1

    g0 = jnp.where(keep0 > 0, i1 * CP_ + p0c, C_)
    g1 = jnp.where(keep1 > 0, i2 * CP_ + p1c, C_)
    g0_ref[...] = g0.reshape(1, 1, TB_)
    g1_ref[...] = g1.reshape(1, 1, TB_)

    tokf = (pid * TB_ + lax.broadcasted_iota(jnp.int32, (TB_,), 0)).astype(jnp.float32)
    iota_et = lax.broadcasted_iota(jnp.int32, (E_, TB_), 0)
    iota_c = lax.broadcasted_iota(jnp.int32, (TB_, C_), 1)
    oh0t = (iota_et == i1[None, :]).astype(jnp.float32)
    oh1t = (iota_et == i2[None, :]).astype(jnp.float32)
    ohc0 = (iota_c == p0c[:, None]).astype(jnp.float32) * keep0[:, None]
    ohc1 = (iota_c == p1c[:, None]).astype(jnp.float32) * keep1[:, None]
    v0 = (tokf + 1.0) * keep0
    v1 = (tokf + 1.0) * keep1
    hi = lax.Precision.HIGHEST
    accs_ref[...] += (jnp.dot(oh0t * v0[None, :], ohc0, precision=hi,
                              preferred_element_type=jnp.float32)
                      + jnp.dot(oh1t * v1[None, :], ohc1, precision=hi,
                                preferred_element_type=jnp.float32))
    accw_ref[...] += (jnp.dot(oh0t * weff0[None, :], ohc0, precision=hi,
                              preferred_element_type=jnp.float32)
                      + jnp.dot(oh1t * weff1[None, :], ohc1, precision=hi,
                                preferred_element_type=jnp.float32))

    @pl.when(pid == NB_ - 1)
    def _fin():
        s = accs_ref[...]
        srctok_ref[...] = jnp.where(s < 0.5, jnp.float32(T_), s - 1.0).astype(jnp.int32)
        wslot_ref[...] = accw_ref[...]


_ROUTER_CALL_KW = dict(
    grid=(NB_,),
    in_specs=[
        pl.BlockSpec((TB_, D_), lambda i: (i, 0)),
        pl.BlockSpec((D_, R_), lambda i: (0, 0)),
        pl.BlockSpec((1, R_), lambda i: (0, 0)),
        pl.BlockSpec((R_, R_), lambda i: (0, 0)),
        pl.BlockSpec((R_, E_), lambda i: (0, 0)),
        pl.BlockSpec((1, E_), lambda i: (0, 0)),
    ],
    out_specs=[
        pl.BlockSpec((1, 1, TB_), lambda i: (i, 0, 0)),
        pl.BlockSpec((1, 1, TB_), lambda i: (i, 0, 0)),
        pl.BlockSpec((E_, C_), lambda i: (0, 0)),
        pl.BlockSpec((E_, C_), lambda i: (0, 0)),
    ],
    out_shape=[
        jax.ShapeDtypeStruct((NB_, 1, TB_), jnp.int32),
        jax.ShapeDtypeStruct((NB_, 1, TB_), jnp.int32),
        jax.ShapeDtypeStruct((E_, C_), jnp.int32),
        jax.ShapeDtypeStruct((E_, C_), jnp.float32),
    ],
    scratch_shapes=[
        pltpu.VMEM((1, E_), jnp.float32),
        pltpu.VMEM((E_, C_), jnp.float32),
        pltpu.VMEM((E_, C_), jnp.float32),
    ],
)

_router = pl.pallas_call(_router_body, **_ROUTER_CALL_KW)


def _layernorm(x, g, b):
    m = jnp.mean(x, axis=-1, keepdims=True)
    v = jnp.mean((x - m) ** 2, axis=-1, keepdims=True)
    return (x - m) / jnp.sqrt(v + 1e-5) * g + b


def _expert_body(stok_ref, x_hbm, st_ref, ws_ref, emb_ref, wqkv_ref, bqkv_ref,
                 wo_ref, bo_ref, w1_ref, b1_ref, w2_ref, b2_ref,
                 g1_ref, be1_ref, g2_ref, be2_ref, out_ref, xbuf, sems):
    e = pl.program_id(0)
    slot = lax.rem(e, 2)

    def _issue(ee, sl):
        base = ee * C_
        for j in range(C_):
            tok = jnp.minimum(stok_ref[base + j], T_ - 1)
            pltpu.make_async_copy(
                x_hbm.at[pl.ds(tok, 1), :],
                xbuf.at[sl, pl.ds(j, 1), :],
                sems.at[sl],
            ).start()

    @pl.when(e == 0)
    def _first():
        _issue(0, 0)

    @pl.when(e + 1 < E_)
    def _next():
        _issue(e + 1, lax.rem(e + 1, 2))

    pltpu.make_async_copy(
        x_hbm.at[pl.ds(0, C_), :], xbuf.at[slot], sems.at[slot]
    ).wait()
    xin = xbuf[slot]
    vmask = st_ref[0] != T_
    kbias = jnp.where(vmask, jnp.float32(0.0), jnp.float32(-1e9))
    xe = xin + emb_ref[0]
    qkv = jnp.dot(xe, wqkv_ref[0], preferred_element_type=jnp.float32) + bqkv_ref[0]
    scale = jnp.float32(1.0) / jnp.sqrt(jnp.float32(DH_))
    qs = qkv[:, 0:D_] * scale
    cols = []
    for hh in range(H_):
        qh = qs[:, hh * DH_:(hh + 1) * DH_]
        kh = qkv[:, D_ + hh * DH_:D_ + (hh + 1) * DH_]
        vh = qkv[:, 2 * D_ + hh * DH_:2 * D_ + (hh + 1) * DH_]
        s = lax.dot_general(qh, kh, (((1,), (1,)), ((), ())),
                            preferred_element_type=jnp.float32) + kbias
        e = jnp.exp(s)
        rec = 1.0 / jnp.sum(e, axis=1, keepdims=True)
        cols.append(jnp.dot(e, vh, preferred_element_type=jnp.float32) * rec)
    ctx = jnp.concatenate(cols, axis=1)
    attn = jnp.dot(ctx, wo_ref[0], preferred_element_type=jnp.float32) + bo_ref[0]
    x1 = _layernorm(xe + attn, g1_ref[0], be1_ref[0])
    hdn = jnp.dot(x1, w1_ref[0], preferred_element_type=jnp.float32) + b1_ref[0]
    hdn = 0.5 * hdn * (1.0 + lax.erf(hdn * jnp.float32(0.7071067811865476)))
    ffn = jnp.dot(hdn, w2_ref[0], preferred_element_type=jnp.float32) + b2_ref[0]
    y = _layernorm(x1 + ffn, g2_ref[0], be2_ref[0])
    out_ref[0, 0:C_, :] = y * ws_ref[0]
    out_ref[0, C_:CP_, :] = jnp.zeros((CP_ - C_, D_), jnp.float32)


_EXPERT_CALL_KW = dict(
    grid_spec=pltpu.PrefetchScalarGridSpec(
        num_scalar_prefetch=1,
        grid=(E_,),
        in_specs=[
            pl.BlockSpec(memory_space=pl.ANY),
            pl.BlockSpec((1, 1, C_), lambda e, s: (e, 0, 0)),
            pl.BlockSpec((1, C_, 1), lambda e, s: (e, 0, 0)),
            pl.BlockSpec((1, 1, D_), lambda e, s: (e, 0, 0)),
            pl.BlockSpec((1, D_, 3 * D_), lambda e, s: (e, 0, 0)),
            pl.BlockSpec((1, 1, 3 * D_), lambda e, s: (e, 0, 0)),
            pl.BlockSpec((1, D_, D_), lambda e, s: (e, 0, 0)),
            pl.BlockSpec((1, 1, D_), lambda e, s: (e, 0, 0)),
            pl.BlockSpec((1, D_, FFN_), lambda e, s: (e, 0, 0)),
            pl.BlockSpec((1, 1, FFN_), lambda e, s: (e, 0, 0)),
            pl.BlockSpec((1, FFN_, D_), lambda e, s: (e, 0, 0)),
            pl.BlockSpec((1, 1, D_), lambda e, s: (e, 0, 0)),
            pl.BlockSpec((1, 1, D_), lambda e, s: (e, 0, 0)),
            pl.BlockSpec((1, 1, D_), lambda e, s: (e, 0, 0)),
            pl.BlockSpec((1, 1, D_), lambda e, s: (e, 0, 0)),
            pl.BlockSpec((1, 1, D_), lambda e, s: (e, 0, 0)),
        ],
        out_specs=pl.BlockSpec((1, CP_, D_), lambda e, s: (e, 0, 0)),
        scratch_shapes=[
            pltpu.VMEM((2, C_, D_), jnp.float32),
            pltpu.SemaphoreType.DMA((2,)),
        ],
    ),
    out_shape=jax.ShapeDtypeStruct((E_, CP_, D_), jnp.float32),
)

_expert = pl.pallas_call(_expert_body, **_EXPERT_CALL_KW)


_NC = 2
_NW = _NC * 16
_CROWS = T_ // _NW

@functools.cache
def _sc_kernels():
    mesh = plsc.VectorSubcoreMesh(core_axis_name="c", subcore_axis_name="s")

    @functools.partial(
        pl.kernel,
        mesh=mesh,
        out_type=jax.ShapeDtypeStruct((T_, D_), jnp.float32),
        scratch_types=[
            pltpu.VMEM((_CROWS,), jnp.int32),
            pltpu.VMEM((_CROWS,), jnp.int32),
            pltpu.VMEM((_CROWS, D_), jnp.float32),
            pltpu.VMEM((_CROWS, D_), jnp.float32),
            pltpu.SemaphoreType.DMA,
            pltpu.SemaphoreType.DMA,
        ],
    )
    def _combine(eo_hbm, g0_hbm, g1_hbm, out_hbm, i0_v, i1_v, r0_v, r1_v, g0s, g1s):
        wid = lax.axis_index("s") * _NC + lax.axis_index("c")
        b = wid * _CROWS
        pltpu.sync_copy(g0_hbm.at[pl.ds(b, _CROWS)], i0_v)
        pltpu.sync_copy(g1_hbm.at[pl.ds(b, _CROWS)], i1_v)
        cp0 = pltpu.async_copy(eo_hbm.at[i0_v], r0_v, g0s)
        cp1 = pltpu.async_copy(eo_hbm.at[i1_v], r1_v, g1s)
        cp0.wait()
        cp1.wait()

        def _add_row(i, carry):
            for j in range(D_ // 16):
                sl = pl.ds(j * 16, 16)
                r0_v[i, sl] = r0_v[i, sl] + r1_v[i, sl]
            return carry

        lax.fori_loop(0, _CROWS, _add_row, 0)
        pltpu.sync_copy(r0_v, out_hbm.at[pl.ds(b, _CROWS)])

    return _combine


def kernel(x, W_in, b_in, tau, A, Bmat, W_gate, b_gate, expert_embed,
           Wqkv, bqkv, Wo, bo, W1, b1, W2, b2, ln1_g, ln1_b, ln2_g, ln2_b):
    x_flat = x.reshape(T_, D_)
    g0b, g1b, src_tok, w_slot = _router(
        x_flat, W_in, b_in.reshape(1, R_), Bmat, W_gate, b_gate.reshape(1, E_))
    g0 = g0b.reshape(T_)
    g1 = g1b.reshape(T_)
    _combine = _sc_kernels()
    exp_out = _expert(
        src_tok.reshape(E_ * C_),
        x_flat,
        src_tok.reshape(E_, 1, C_),
        w_slot.reshape(E_, C_, 1),
        expert_embed.reshape(E_, 1, D_),
        Wqkv, bqkv.reshape(E_, 1, 3 * D_),
        Wo, bo.reshape(E_, 1, D_),
        W1, b1.reshape(E_, 1, FFN_),
        W2, b2.reshape(E_, 1, D_),
        ln1_g.reshape(E_, 1, D_), ln1_b.reshape(E_, 1, D_),
        ln2_g.reshape(E_, 1, D_), ln2_b.reshape(E_, 1, D_),
    )
    y = _combine(exp_out.reshape(E_ * CP_, D_), g0, g1)
    return y.reshape(B_, S_, D_)


# speedup vs baseline: 1.0007x; 1.0007x over previous
import functools

import jax
import jax.numpy as jnp
from jax import lax
from jax.experimental import pallas as pl
from jax.experimental.pallas import tpu as pltpu
from jax.experimental.pallas import tpu_sc as plsc

B_, S_, D_, R_ = 1, 2048, 768, 256
E_, K_, H_ = 64, 2, 32
DH_ = D_ // H_
FFN_ = 3072
T_ = B_ * S_
C_ = 80
CP_ = 88
TB_ = 128
NB_ = T_ // TB_


def _router_body(x_ref, win_ref, bin_ref, bmat_ref, wg_ref, bg_ref,
                 g0_ref, g1_ref, srctok_ref, wslot_ref,
                 carry_ref, accs_ref, accw_ref):
    pid = pl.program_id(0)

    @pl.when(pid == 0)
    def _init():
        carry_ref[...] = jnp.zeros_like(carry_ref)
        accs_ref[...] = jnp.zeros_like(accs_ref)
        accw_ref[...] = jnp.zeros_like(accw_ref)

    xb = x_ref[...]
    xp = jnp.dot(xb, win_ref[...], preferred_element_type=jnp.float32) + bin_ref[...]
    h = 0.1 * jnp.tanh(jnp.dot(xp, bmat_ref[...], preferred_element_type=jnp.float32))
    logits = jnp.dot(h, wg_ref[...], preferred_element_type=jnp.float32) + bg_ref[...]

    iota_e = lax.broadcasted_iota(jnp.int32, (TB_, E_), 1)
    m1 = jnp.max(logits, axis=1, keepdims=True)
    i1 = jnp.min(jnp.where(logits == m1, iota_e, E_), axis=1)
    masked = jnp.where(iota_e == i1[:, None], jnp.float32(-jnp.inf), logits)
    m2 = jnp.max(masked, axis=1, keepdims=True)
    i2 = jnp.min(jnp.where(masked == m2, iota_e, E_), axis=1)

    t = jnp.exp(m2 - m1)
    wa = 1.0 / (1.0 + t)
    wb = t / (1.0 + t)

    oh0 = (iota_e == i1[:, None]).astype(jnp.float32)
    oh1 = (iota_e == i2[:, None]).astype(jnp.float32)
    both = oh0 + oh1
    r_i = lax.broadcasted_iota(jnp.int32, (TB_, TB_), 0)
    c_i = lax.broadcasted_iota(jnp.int32, (TB_, TB_), 1)
    ltri = (c_i < r_i).astype(jnp.float32)
    sexc = jnp.dot(ltri, both, preferred_element_type=jnp.float32)
    base = carry_ref[...] + sexc
    pos0 = jnp.sum(oh0 * base, axis=1)
    pos1 = jnp.sum(oh1 * base, axis=1)
    carry_ref[...] = carry_ref[...] + jnp.sum(both, axis=0, keepdims=True)

    keep0 = (pos0 < C_).astype(jnp.float32)
    keep1 = (pos1 < C_).astype(jnp.float32)
    p0c = jnp.minimum(pos0, C_ - 1).astype(jnp.int32)
    p1c = jnp.minimum(pos1, C_ - 1).astype(jnp.int32)
    weff0 = wa[:, 0] * keep0
    weff1 = wb[:, 0] * keep1

    g0 = jnp.where(keep0 > 0, i1 * CP_ + p0c, C_)
    g1 = jnp.where(keep1 > 0, i2 * CP_ + p1c, C_)
    g0_ref[...] = g0.reshape(1, 1, TB_)
    g1_ref[...] = g1.reshape(1, 1, TB_)

    tokf = (pid * TB_ + lax.broadcasted_iota(jnp.int32, (TB_,), 0)).astype(jnp.float32)
    iota_et = lax.broadcasted_iota(jnp.int32, (E_, TB_), 0)
    iota_c = lax.broadcasted_iota(jnp.int32, (TB_, C_), 1)
    oh0t = (iota_et == i1[None, :]).astype(jnp.float32)
    oh1t = (iota_et == i2[None, :]).astype(jnp.float32)
    ohc0 = (iota_c == p0c[:, None]).astype(jnp.float32) * keep0[:, None]
    ohc1 = (iota_c == p1c[:, None]).astype(jnp.float32) * keep1[:, None]
    v0 = (tokf + 1.0) * keep0
    v1 = (tokf + 1.0) * keep1
    hi = lax.Precision.HIGHEST
    accs_ref[...] += (jnp.dot(oh0t * v0[None, :], ohc0, precision=hi,
                              preferred_element_type=jnp.float32)
                      + jnp.dot(oh1t * v1[None, :], ohc1, precision=hi,
                                preferred_element_type=jnp.float32))
    accw_ref[...] += (jnp.dot(oh0t * weff0[None, :], ohc0, precision=hi,
                              preferred_element_type=jnp.float32)
                      + jnp.dot(oh1t * weff1[None, :], ohc1, precision=hi,
                                preferred_element_type=jnp.float32))

    @pl.when(pid == NB_ - 1)
    def _fin():
        s = accs_ref[...]
        srctok_ref[...] = jnp.where(s < 0.5, jnp.float32(T_), s - 1.0).astype(jnp.int32)
        wslot_ref[...] = accw_ref[...]


_ROUTER_CALL_KW = dict(
    grid=(NB_,),
    in_specs=[
        pl.BlockSpec((TB_, D_), lambda i: (i, 0)),
        pl.BlockSpec((D_, R_), lambda i: (0, 0)),
        pl.BlockSpec((1, R_), lambda i: (0, 0)),
        pl.BlockSpec((R_, R_), lambda i: (0, 0)),
        pl.BlockSpec((R_, E_), lambda i: (0, 0)),
        pl.BlockSpec((1, E_), lambda i: (0, 0)),
    ],
    out_specs=[
        pl.BlockSpec((1, 1, TB_), lambda i: (i, 0, 0)),
        pl.BlockSpec((1, 1, TB_), lambda i: (i, 0, 0)),
        pl.BlockSpec((E_, C_), lambda i: (0, 0)),
        pl.BlockSpec((E_, C_), lambda i: (0, 0)),
    ],
    out_shape=[
        jax.ShapeDtypeStruct((NB_, 1, TB_), jnp.int32),
        jax.ShapeDtypeStruct((NB_, 1, TB_), jnp.int32),
        jax.ShapeDtypeStruct((E_, C_), jnp.int32),
        jax.ShapeDtypeStruct((E_, C_), jnp.float32),
    ],
    scratch_shapes=[
        pltpu.VMEM((1, E_), jnp.float32),
        pltpu.VMEM((E_, C_), jnp.float32),
        pltpu.VMEM((E_, C_), jnp.float32),
    ],
)

_router = pl.pallas_call(_router_body, **_ROUTER_CALL_KW)


def _layernorm(x, g, b):
    m = jnp.mean(x, axis=-1, keepdims=True)
    v = jnp.mean((x - m) ** 2, axis=-1, keepdims=True)
    return (x - m) / jnp.sqrt(v + 1e-5) * g + b


def _expert_body(stok_ref, x_hbm, st_ref, ws_ref, emb_ref, wqkv_ref, bqkv_ref,
                 wo_ref, bo_ref, w1_ref, b1_ref, w2_ref, b2_ref,
                 g1_ref, be1_ref, g2_ref, be2_ref, out_ref, xbuf, sems):
    e = pl.program_id(0)
    slot = lax.rem(e, 2)

    def _issue(ee, sl):
        base = ee * C_
        for j in range(C_):
            tok = jnp.minimum(stok_ref[base + j], T_ - 1)
            pltpu.make_async_copy(
                x_hbm.at[pl.ds(tok, 1), :],
                xbuf.at[sl, pl.ds(j, 1), :],
                sems.at[sl],
            ).start()

    @pl.when(e == 0)
    def _first():
        _issue(0, 0)

    @pl.when(e + 1 < E_)
    def _next():
        _issue(e + 1, lax.rem(e + 1, 2))

    pltpu.make_async_copy(
        x_hbm.at[pl.ds(0, C_), :], xbuf.at[slot], sems.at[slot]
    ).wait()
    xin = xbuf[slot]
    vmask = st_ref[0] != T_
    kbias = jnp.where(vmask, jnp.float32(0.0), jnp.float32(-1e9))
    xe = xin + emb_ref[0]
    qkv = jnp.dot(xe, wqkv_ref[0], preferred_element_type=jnp.float32) + bqkv_ref[0]
    scale = jnp.float32(1.0) / jnp.sqrt(jnp.float32(DH_))
    qs = qkv[:, 0:D_] * scale
    cols = []
    for hh in range(H_):
        qh = qs[:, hh * DH_:(hh + 1) * DH_]
        kh = qkv[:, D_ + hh * DH_:D_ + (hh + 1) * DH_]
        vh = qkv[:, 2 * D_ + hh * DH_:2 * D_ + (hh + 1) * DH_]
        s = lax.dot_general(qh, kh, (((1,), (1,)), ((), ())),
                            preferred_element_type=jnp.float32) + kbias
        e = jnp.exp(s)
        rec = 1.0 / jnp.sum(e, axis=1, keepdims=True)
        cols.append(jnp.dot(e, vh, preferred_element_type=jnp.float32) * rec)
    ctx = jnp.concatenate(cols, axis=1)
    attn = jnp.dot(ctx, wo_ref[0], preferred_element_type=jnp.float32) + bo_ref[0]
    x1 = _layernorm(xe + attn, g1_ref[0], be1_ref[0])
    hdn = jnp.dot(x1, w1_ref[0], preferred_element_type=jnp.float32) + b1_ref[0]
    hdn = 0.5 * hdn * (1.0 + lax.erf(hdn * jnp.float32(0.7071067811865476)))
    ffn = jnp.dot(hdn, w2_ref[0], preferred_element_type=jnp.float32) + b2_ref[0]
    y = _layernorm(x1 + ffn, g2_ref[0], be2_ref[0])
    out_ref[0, 0:C_, :] = y * ws_ref[0]
    out_ref[0, C_:CP_, :] = jnp.zeros((CP_ - C_, D_), jnp.float32)


_EXPERT_CALL_KW = dict(
    grid_spec=pltpu.PrefetchScalarGridSpec(
        num_scalar_prefetch=1,
        grid=(E_,),
        in_specs=[
            pl.BlockSpec(memory_space=pl.ANY),
            pl.BlockSpec((1, 1, C_), lambda e, s: (e, 0, 0)),
            pl.BlockSpec((1, C_, 1), lambda e, s: (e, 0, 0)),
            pl.BlockSpec((1, 1, D_), lambda e, s: (e, 0, 0)),
            pl.BlockSpec((1, D_, 3 * D_), lambda e, s: (e, 0, 0)),
            pl.BlockSpec((1, 1, 3 * D_), lambda e, s: (e, 0, 0)),
            pl.BlockSpec((1, D_, D_), lambda e, s: (e, 0, 0)),
            pl.BlockSpec((1, 1, D_), lambda e, s: (e, 0, 0)),
            pl.BlockSpec((1, D_, FFN_), lambda e, s: (e, 0, 0)),
            pl.BlockSpec((1, 1, FFN_), lambda e, s: (e, 0, 0)),
            pl.BlockSpec((1, FFN_, D_), lambda e, s: (e, 0, 0)),
            pl.BlockSpec((1, 1, D_), lambda e, s: (e, 0, 0)),
            pl.BlockSpec((1, 1, D_), lambda e, s: (e, 0, 0)),
            pl.BlockSpec((1, 1, D_), lambda e, s: (e, 0, 0)),
            pl.BlockSpec((1, 1, D_), lambda e, s: (e, 0, 0)),
            pl.BlockSpec((1, 1, D_), lambda e, s: (e, 0, 0)),
        ],
        out_specs=pl.BlockSpec((1, CP_, D_), lambda e, s: (e, 0, 0)),
        scratch_shapes=[
            pltpu.VMEM((2, C_, D_), jnp.float32),
            pltpu.SemaphoreType.DMA((2,)),
        ],
    ),
    out_shape=jax.ShapeDtypeStruct((E_, CP_, D_), jnp.float32),
    compiler_params=pltpu.CompilerParams(vmem_limit_bytes=100 * 1024 * 1024),
)

_expert = pl.pallas_call(_expert_body, **_EXPERT_CALL_KW)


_NC = 2
_NW = _NC * 16
_CROWS = T_ // _NW

@functools.cache
def _sc_kernels():
    mesh = plsc.VectorSubcoreMesh(core_axis_name="c", subcore_axis_name="s")

    @functools.partial(
        pl.kernel,
        mesh=mesh,
        out_type=jax.ShapeDtypeStruct((T_, D_), jnp.float32),
        scratch_types=[
            pltpu.VMEM((_CROWS,), jnp.int32),
            pltpu.VMEM((_CROWS,), jnp.int32),
            pltpu.VMEM((_CROWS, D_), jnp.float32),
            pltpu.VMEM((_CROWS, D_), jnp.float32),
            pltpu.SemaphoreType.DMA,
            pltpu.SemaphoreType.DMA,
        ],
    )
    def _combine(eo_hbm, g0_hbm, g1_hbm, out_hbm, i0_v, i1_v, r0_v, r1_v, g0s, g1s):
        wid = lax.axis_index("s") * _NC + lax.axis_index("c")
        b = wid * _CROWS
        pltpu.sync_copy(g0_hbm.at[pl.ds(b, _CROWS)], i0_v)
        pltpu.sync_copy(g1_hbm.at[pl.ds(b, _CROWS)], i1_v)
        cp0 = pltpu.async_copy(eo_hbm.at[i0_v], r0_v, g0s)
        cp1 = pltpu.async_copy(eo_hbm.at[i1_v], r1_v, g1s)
        cp0.wait()
        cp1.wait()

        def _add_row(i, carry):
            for j in range(D_ // 16):
                sl = pl.ds(j * 16, 16)
                r0_v[i, sl] = r0_v[i, sl] + r1_v[i, sl]
            return carry

        lax.fori_loop(0, _CROWS, _add_row, 0)
        pltpu.sync_copy(r0_v, out_hbm.at[pl.ds(b, _CROWS)])

    return _combine


def kernel(x, W_in, b_in, tau, A, Bmat, W_gate, b_gate, expert_embed,
           Wqkv, bqkv, Wo, bo, W1, b1, W2, b2, ln1_g, ln1_b, ln2_g, ln2_b):
    x_flat = x.reshape(T_, D_)
    g0b, g1b, src_tok, w_slot = _router(
        x_flat, W_in, b_in.reshape(1, R_), Bmat, W_gate, b_gate.reshape(1, E_))
    g0 = g0b.reshape(T_)
    g1 = g1b.reshape(T_)
    _combine = _sc_kernels()
    exp_out = _expert(
        src_tok.reshape(E_ * C_),
        x_flat,
        src_tok.reshape(E_, 1, C_),
        w_slot.reshape(E_, C_, 1),
        expert_embed.reshape(E_, 1, D_),
        Wqkv, bqkv.reshape(E_, 1, 3 * D_),
        Wo, bo.reshape(E_, 1, D_),
        W1, b1.reshape(E_, 1, FFN_),
        W2, b2.reshape(E_, 1, D_),
        ln1_g.reshape(E_, 1, D_), ln1_b.reshape(E_, 1, D_),
        ln2_g.reshape(E_, 1, D_), ln2_b.reshape(E_, 1, D_),
    )
    y = _combine(exp_out.reshape(E_ * CP_, D_), g0, g1)
    return y.reshape(B_, S_, D_)
